# Initial kernel scaffold; baseline (speedup 1.0000x reference)
import jax
import jax.numpy as jnp
from jax.experimental import pallas as pl


def kernel(x1, edge_attr1, x2, edge_attr2, params, edge_index1, edge_index2):
    raise NotImplementedError("write your pallas kernel here")


# speedup vs baseline: 1.6535x; 1.6535x over previous
import functools

import jax
import jax.numpy as jnp
from jax.experimental import pallas as pl
from jax.experimental.pallas import tpu as pltpu

N1 = 10000
N2 = 10000
E = 160000
K_NEAR = 10

_ROWS = 256
_COLS_PAD = 10112
_N1_PAD = 10240


def _knn_body(x1_ref, x2t_ref, out_ref, d_ref):
    a = x1_ref[...]
    b = x2t_ref[...]
    ab = jax.lax.dot_general(a, b, (((1,), (0,)), ((), ())),
                             preferred_element_type=jnp.float32)
    aa = jnp.sum(a * a, axis=1, keepdims=True)
    bb = jnp.sum(b * b, axis=0, keepdims=True)
    d2 = aa + bb - 2.0 * ab
    d = jnp.sqrt(jnp.maximum(d2, 0.0))
    cols = jax.lax.broadcasted_iota(jnp.int32, (_ROWS, _COLS_PAD), 1)
    d = jnp.where(cols >= N2, jnp.inf, d)
    d_ref[...] = d
    for k in range(K_NEAR):
        dcur = d_ref[...]
        m = jnp.min(dcur, axis=1, keepdims=True)
        idx = jnp.min(jnp.where(dcur <= m, cols, jnp.int32(2 ** 30)),
                      axis=1, keepdims=True)
        out_ref[:, k:k + 1] = idx
        d_ref[...] = jnp.where(cols == idx, jnp.inf, dcur)


def _knn(x1, x2):
    a = jnp.zeros((_N1_PAD, 8), jnp.float32).at[:N1, :3].set(x1[:, :3])
    bt = jnp.zeros((8, _COLS_PAD), jnp.float32).at[:3, :N2].set(x2[:, :3].T)
    grid = _N1_PAD // _ROWS
    out = pl.pallas_call(
        _knn_body,
        grid=(grid,),
        in_specs=[
            pl.BlockSpec((_ROWS, 8), lambda i: (i, 0)),
            pl.BlockSpec((8, _COLS_PAD), lambda i: (0, 0)),
        ],
        out_specs=pl.BlockSpec((_ROWS, 16), lambda i: (i, 0)),
        out_shape=jax.ShapeDtypeStruct((_N1_PAD, 16), jnp.int32),
        scratch_shapes=[pltpu.VMEM((_ROWS, _COLS_PAD), jnp.float32)],
    )(a, bt)
    return out[:N1, :K_NEAR]


_PBLK = 2000


def _pair_body(d_ref, w1_ref, b1_ref, w2_ref, b2_ref, mw_ref, mb_ref,
               m2w_ref, m2b_ref, out_ref):
    d = d_ref[...]
    acc = jnp.zeros_like(d)
    for j in range(32):
        acc = acc + jnp.maximum(d * w1_ref[0, j] + b1_ref[0, j], 0.0) * w2_ref[0, j]
    t = acc + b2_ref[0, 0]
    t1 = jax.lax.dot_general(t, mw_ref[...], (((1,), (1,)), ((), ())),
                             preferred_element_type=jnp.float32) + mb_ref[...]
    t2 = jax.lax.dot_general(t1, m2w_ref[...], (((1,), (1,)), ((), ())),
                             preferred_element_type=jnp.float32) + m2b_ref[...]
    out_ref[...] = t2


def _pair_mlp(diff, mlp, m_w, m_b, m2_w, m2_b):
    P = diff.shape[0]
    grid = P // _PBLK
    small = lambda r, c: pl.BlockSpec((r, c), lambda i: (0, 0))
    return pl.pallas_call(
        _pair_body,
        grid=(grid,),
        in_specs=[
            pl.BlockSpec((_PBLK, 32), lambda i: (i, 0)),
            small(1, 32), small(1, 32), small(1, 32), small(1, 1),
            small(10, 32), small(1, 10), small(2, 10), small(1, 2),
        ],
        out_specs=pl.BlockSpec((_PBLK, 2), lambda i: (i, 0)),
        out_shape=jax.ShapeDtypeStruct((P, 2), jnp.float32),
    )(diff, mlp["W1"].reshape(1, 32), mlp["b1"].reshape(1, 32),
      mlp["W2"].reshape(1, 32), mlp["b2"].reshape(1, 1),
      m_w, m_b.reshape(1, 10), m2_w, m2_b.reshape(1, 2))


def _gat(p, x, src, dst, n):
    h = x @ p["W"]
    a_src = h @ p["att_src"]
    a_dst = h @ p["att_dst"]
    alpha = jax.nn.leaky_relu(a_src[src] + a_dst[dst], negative_slope=0.2)
    amax = jax.ops.segment_max(alpha, dst, num_segments=n)
    ex = jnp.exp(alpha - amax[dst])
    denom = jax.ops.segment_sum(ex, dst, num_segments=n)
    coef = ex / denom[dst]
    out = jax.ops.segment_sum(coef[:, None] * h[src], dst, num_segments=n)
    return out + p["bias"]


def _dgcnn(enc, x, src, dst, n_layers):
    n = x.shape[0]
    xs = [x]
    for p in enc["layers"][:n_layers]:
        xs.append(_gat(p, xs[-1], src, dst, n))
    xcat = jnp.concatenate(xs, axis=-1)
    w = enc["conv_w"]
    ksize = w.shape[2]
    return xcat[:, :ksize] @ w[:, 0, :].T + enc["conv_b"]


def _each_graph(params, x, edge_index):
    n = x.shape[0]
    loop = jnp.arange(n, dtype=edge_index.dtype)
    src = jnp.concatenate([edge_index[0], loop])
    dst = jnp.concatenate([edge_index[1], loop])
    emb = _dgcnn(params["enc1"], x, src, dst, 5)
    emb2 = _dgcnn(params["enc2"], x, src, dst, 2)
    return jnp.concatenate([emb2, emb], axis=1)


def kernel(x1, edge_attr1, x2, edge_attr2, params, edge_index1, edge_index2):
    pred1 = _each_graph(params, x1, edge_index1)
    pred2 = _each_graph(params, x2, edge_index2)
    nearest = _knn(x1, x2)
    cols = jnp.sort(nearest, axis=1).reshape(-1)
    rows = jnp.repeat(jnp.arange(N1), K_NEAR)
    diff = jnp.abs(pred1[rows] - pred2[cols])
    return _pair_mlp(diff, params["mlp"], params["m_w"], params["m_b"],
                     params["m2_w"], params["m2_b"])


# speedup vs baseline: 27.2847x; 16.5009x over previous
import functools

import jax
import jax.numpy as jnp
from jax import lax
from jax.experimental import pallas as pl
from jax.experimental.pallas import tpu as pltpu
from jax.experimental.pallas import tpu_sc as plsc

N1 = 10000
N2 = 10000
E = 160000
K_NEAR = 10

_NP = 10048
_EPG = 171008
_CH = _EPG // 32
_NPAIR = _CH // 32
_RPT = _NP // 16
_B = 1256


def _make_edge_kernel(F, FA):
    mesh = plsc.VectorSubcoreMesh(core_axis_name="c", subcore_axis_name="s")

    @functools.partial(
        pl.kernel,
        out_type=jax.ShapeDtypeStruct((2, _NP, FA), jnp.float32),
        mesh=mesh,
        scratch_types=[
            pltpu.VMEM((_CH,), jnp.int32),
            pltpu.VMEM((_CH,), jnp.int32),
            pltpu.VMEM((_NP,), jnp.float32),
            pltpu.VMEM((_NP,), jnp.float32),
            pltpu.VMEM((16, F), jnp.float32),
            pltpu.VMEM((16, F), jnp.float32),
            pltpu.VMEM((16, FA), jnp.float32),
            pltpu.VMEM((16, FA), jnp.float32),
            pltpu.VMEM_SHARED((_NP, FA), jnp.float32),
            pltpu.SemaphoreType.DMA,
            pltpu.SemaphoreType.DMA,
        ],
        compiler_params=pltpu.CompilerParams(needs_layout_passes=False,
                                             use_tc_tiling_on_sc=False),
    )
    def k(h_hbm, asrc_hbm, adst_hbm, src_hbm, dst_hbm, zeros_hbm, acc_hbm,
          src_v, dst_v, asrc_v, adst_v, rows0, rows1, sbuf0, sbuf1,
          acc_sh, gsem0, gsem1):
        cid = lax.axis_index("c")
        sid = lax.axis_index("s")
        base = (cid * 16 + sid) * _CH
        pltpu.sync_copy(src_hbm.at[pl.ds(base, _CH)], src_v)
        pltpu.sync_copy(dst_hbm.at[pl.ds(base, _CH)], dst_v)
        pltpu.sync_copy(asrc_hbm, asrc_v)
        pltpu.sync_copy(adst_hbm, adst_v)
        ro = sid * _RPT
        pltpu.sync_copy(zeros_hbm.at[pl.ds(ro, _RPT)], acc_sh.at[pl.ds(ro, _RPT)])
        plsc.subcore_barrier()

        bufs = ((rows0, sbuf0, gsem0), (rows1, sbuf1, gsem1))
        lane0 = lax.iota(jnp.int32, 16) == 0

        def pair_body(p, carry):
            g0 = p * 2
            idxs, cps = [], []
            for b in range(2):
                i_s = src_v[pl.ds((g0 + b) * 16, 16)]
                cps.append(pltpu.async_copy(h_hbm.at[i_s], bufs[b][0], bufs[b][2]))
                idxs.append(i_s)
            for b in range(2):
                rows, sbuf, _ = bufs[b]
                i_d = dst_v[pl.ds((g0 + b) * 16, 16)]
                a = plsc.load_gather(asrc_v, [idxs[b]]) + plsc.load_gather(adst_v, [i_d])
                alpha = jnp.where(a >= 0.0, a, 0.2 * a)
                w = jnp.exp(alpha)
                cps[b].wait()
                for e in range(16):
                    we = w[e]
                    for q in range(F // 16):
                        sbuf[e, pl.ds(q * 16, 16)] = rows[e, pl.ds(q * 16, 16)] * we
                    sbuf[e, pl.ds(F, 16)] = jnp.where(lane0, we, 0.0)
                pltpu.sync_copy(sbuf, acc_sh.at[i_d], add=True)
            return carry

        lax.fori_loop(0, _NPAIR, pair_body, 0)
        plsc.subcore_barrier()
        pltpu.sync_copy(acc_sh.at[pl.ds(ro, _RPT)], acc_hbm.at[cid, pl.ds(ro, _RPT)])

    return k


_edge64 = _make_edge_kernel(64, 80)
_edge16 = _make_edge_kernel(16, 32)


def _dot(a, b):
    return lax.dot_general(a, b, (((1,), (0,)), ((), ())),
                           preferred_element_type=jnp.float32)


def _mm_att_body(x_ref, w_ref, as_ref, ad_ref, h_ref, s_ref, d_ref):
    h = _dot(x_ref[...], w_ref[...])
    h_ref[...] = h
    s_ref[...] = _dot(h, as_ref[...])
    d_ref[...] = _dot(h, ad_ref[...])


def _mm_att(x, W, att_s, att_d):
    fin, F = W.shape
    h, a_s, a_d = pl.pallas_call(
        _mm_att_body,
        grid=(_NP // _B,),
        in_specs=[
            pl.BlockSpec((_B, fin), lambda i: (i, 0)),
            pl.BlockSpec((fin, F), lambda i: (0, 0)),
            pl.BlockSpec((F, 1), lambda i: (0, 0)),
            pl.BlockSpec((F, 1), lambda i: (0, 0)),
        ],
        out_specs=[
            pl.BlockSpec((_B, F), lambda i: (i, 0)),
            pl.BlockSpec((_B, 1), lambda i: (i, 0)),
            pl.BlockSpec((_B, 1), lambda i: (i, 0)),
        ],
        out_shape=[
            jax.ShapeDtypeStruct((_NP, F), jnp.float32),
            jax.ShapeDtypeStruct((_NP, 1), jnp.float32),
            jax.ShapeDtypeStruct((_NP, 1), jnp.float32),
        ],
    )(x, W, att_s, att_d)
    return h, a_s.reshape(_NP), a_d.reshape(_NP)


def _combine(acc, bias_p, F_tbl):
    FA = acc.shape[2]

    def body(acc_ref, b_ref, out_ref):
        a = acc_ref[0] + acc_ref[1]
        out_ref[...] = a[:, :F_tbl] / a[:, F_tbl:F_tbl + 1] + b_ref[...]

    return pl.pallas_call(
        body,
        grid=(_NP // _B,),
        in_specs=[
            pl.BlockSpec((2, _B, FA), lambda i: (0, i, 0)),
            pl.BlockSpec((1, F_tbl), lambda i: (0, 0)),
        ],
        out_specs=pl.BlockSpec((_B, F_tbl), lambda i: (i, 0)),
        out_shape=jax.ShapeDtypeStruct((_NP, F_tbl), jnp.float32),
    )(acc, bias_p)


def _final_body(x1_ref, w1_ref, b1_ref, x2_ref, w2_ref, b2_ref, out_ref):
    emb = _dot(x1_ref[...], w1_ref[...]) + b1_ref[...]
    emb2 = _dot(x2_ref[...], w2_ref[...]) + b2_ref[...]
    out_ref[...] = jnp.concatenate([emb2, emb], axis=1)


def _final(xc1, w1p, b1, xc2, w2p, b2):
    return pl.pallas_call(
        _final_body,
        grid=(_NP // _B,),
        in_specs=[
            pl.BlockSpec((_B, 384), lambda i: (i, 0)),
            pl.BlockSpec((384, 16), lambda i: (0, 0)),
            pl.BlockSpec((1, 16), lambda i: (0, 0)),
            pl.BlockSpec((_B, 32), lambda i: (i, 0)),
            pl.BlockSpec((32, 16), lambda i: (0, 0)),
            pl.BlockSpec((1, 16), lambda i: (0, 0)),
        ],
        out_specs=pl.BlockSpec((_B, 32), lambda i: (i, 0)),
        out_shape=jax.ShapeDtypeStruct((_NP, 32), jnp.float32),
    )(xc1, w1p, b1, xc2, w2p, b2)


_ROWS = 256
_COLS_PAD = 10112
_N1_PAD = 10240


def _knn_body(x1_ref, x2t_ref, out_ref, d_ref):
    a = x1_ref[...]
    b = x2t_ref[...]
    ab = _dot(a, b)
    aa = jnp.sum(a * a, axis=1, keepdims=True)
    bb = jnp.sum(b * b, axis=0, keepdims=True)
    d = jnp.sqrt(jnp.maximum(aa + bb - 2.0 * ab, 0.0))
    cols = lax.broadcasted_iota(jnp.int32, (_ROWS, _COLS_PAD), 1)
    d_ref[...] = jnp.where(cols >= N2, jnp.inf, d)
    for k in range(K_NEAR):
        dcur = d_ref[...]
        m = jnp.min(dcur, axis=1, keepdims=True)
        idx = jnp.min(jnp.where(dcur <= m, cols, jnp.int32(2 ** 30)),
                      axis=1, keepdims=True)
        out_ref[:, k:k + 1] = idx
        d_ref[...] = jnp.where(cols == idx, jnp.inf, dcur)


def _knn(x1, x2):
    a = jnp.zeros((_N1_PAD, 8), jnp.float32).at[:N1, :3].set(x1[:, :3])
    bt = jnp.zeros((8, _COLS_PAD), jnp.float32).at[:3, :N2].set(x2[:, :3].T)
    out = pl.pallas_call(
        _knn_body,
        grid=(_N1_PAD // _ROWS,),
        in_specs=[
            pl.BlockSpec((_ROWS, 8), lambda i: (i, 0)),
            pl.BlockSpec((8, _COLS_PAD), lambda i: (0, 0)),
        ],
        out_specs=pl.BlockSpec((_ROWS, 16), lambda i: (i, 0)),
        out_shape=jax.ShapeDtypeStruct((_N1_PAD, 16), jnp.int32),
        scratch_shapes=[pltpu.VMEM((_ROWS, _COLS_PAD), jnp.float32)],
    )(a, bt)
    return out[:N1, :K_NEAR]


_PBLK = 2000


def _pair_body(d_ref, w1_ref, b1_ref, w2_ref, b2_ref, mw_ref, mb_ref,
               m2w_ref, m2b_ref, out_ref):
    d = d_ref[...]
    acc = jnp.zeros_like(d)
    for j in range(32):
        acc = acc + jnp.maximum(d * w1_ref[0, j] + b1_ref[0, j], 0.0) * w2_ref[0, j]
    t = acc + b2_ref[0, 0]
    t1 = lax.dot_general(t, mw_ref[...], (((1,), (1,)), ((), ())),
                         preferred_element_type=jnp.float32) + mb_ref[...]
    t2 = lax.dot_general(t1, m2w_ref[...], (((1,), (1,)), ((), ())),
                         preferred_element_type=jnp.float32) + m2b_ref[...]
    out_ref[...] = t2


def _pair_mlp(diff, mlp, m_w, m_b, m2_w, m2_b):
    P = diff.shape[0]
    small = lambda r, c: pl.BlockSpec((r, c), lambda i: (0, 0))
    return pl.pallas_call(
        _pair_body,
        grid=(P // _PBLK,),
        in_specs=[
            pl.BlockSpec((_PBLK, 32), lambda i: (i, 0)),
            small(1, 32), small(1, 32), small(1, 32), small(1, 1),
            small(10, 32), small(1, 10), small(2, 10), small(1, 2),
        ],
        out_specs=pl.BlockSpec((_PBLK, 2), lambda i: (i, 0)),
        out_shape=jax.ShapeDtypeStruct((P, 2), jnp.float32),
    )(diff, mlp["W1"].reshape(1, 32), mlp["b1"].reshape(1, 32),
      mlp["W2"].reshape(1, 32), mlp["b2"].reshape(1, 1),
      m_w, m_b.reshape(1, 10), m2_w, m2_b.reshape(1, 2))


def _padw(W, r, c):
    return jnp.zeros((r, c), jnp.float32).at[:W.shape[0], :W.shape[1]].set(W)


def _pada(v, n):
    return jnp.zeros((n, 1), jnp.float32).at[:v.shape[0], 0].set(v)


def _padb(v, n):
    return jnp.zeros((1, n), jnp.float32).at[0, :v.shape[0]].set(v)


def _encode(x, edge_index, params, zeros80, zeros32):
    n = x.shape[0]
    xs = jnp.zeros((_NP, 16), jnp.float32).at[:n, :10].set(x)
    loop = jnp.arange(n, dtype=jnp.int32)
    pad_e = jnp.full((_EPG - (E + n),), 10000, jnp.int32)
    src = jnp.concatenate([edge_index[0], loop, pad_e])
    dst = jnp.concatenate([edge_index[1], loop, pad_e])

    e1 = params["enc1"]["layers"]
    e2 = params["enc2"]["layers"]

    outs1 = []
    h, a_s, a_d = _mm_att(xs, _padw(e1[0]["W"], 16, 64),
                          e1[0]["att_src"].reshape(64, 1),
                          e1[0]["att_dst"].reshape(64, 1))
    for i in range(5):
        acc = _edge64(h, a_s, a_d, src, dst, zeros80)
        out = _combine(acc, e1[i]["bias"].reshape(1, 64), 64)
        outs1.append(out)
        if i < 4:
            h, a_s, a_d = _mm_att(out, e1[i + 1]["W"],
                                  e1[i + 1]["att_src"].reshape(64, 1),
                                  e1[i + 1]["att_dst"].reshape(64, 1))

    outs2 = []
    h, a_s, a_d = _mm_att(xs, _padw(e2[0]["W"], 16, 16),
                          _pada(e2[0]["att_src"], 16), _pada(e2[0]["att_dst"], 16))
    for i in range(2):
        acc = _edge16(h, a_s, a_d, src, dst, zeros32)
        out = _combine(acc, _padb(e2[i]["bias"], 16), 16)
        outs2.append(out)
        if i < 1:
            h, a_s, a_d = _mm_att(out, _padw(e2[1]["W"], 16, 16),
                                  _pada(e2[1]["att_src"], 16),
                                  _pada(e2[1]["att_dst"], 16))

    xc1 = jnp.concatenate([xs[:, :10]] + outs1
                          + [jnp.zeros((_NP, 54), jnp.float32)], axis=1)
    w1p = jnp.zeros((384, 16), jnp.float32).at[:321].set(
        params["enc1"]["conv_w"][:, 0, :].T)
    xc2 = jnp.concatenate([xs[:, :10], outs2[0][:, :8], outs2[1][:, :8],
                           jnp.zeros((_NP, 6), jnp.float32)], axis=1)
    w2p = jnp.zeros((32, 16), jnp.float32).at[:25].set(
        params["enc2"]["conv_w"][:, 0, :].T)
    return _final(xc1, w1p, params["enc1"]["conv_b"].reshape(1, 16),
                  xc2, w2p, params["enc2"]["conv_b"].reshape(1, 16))


def kernel(x1, edge_attr1, x2, edge_attr2, params, edge_index1, edge_index2):
    zeros80 = jnp.zeros((_NP, 80), jnp.float32)
    zeros32 = jnp.zeros((_NP, 32), jnp.float32)
    pred1 = _encode(x1, edge_index1, params, zeros80, zeros32)[:N1]
    pred2 = _encode(x2, edge_index2, params, zeros80, zeros32)[:N2]

    nearest = _knn(x1, x2)
    cols = jnp.sort(nearest, axis=1).reshape(-1)
    rows = jnp.repeat(jnp.arange(N1), K_NEAR)
    diff = jnp.abs(pred1[rows] - pred2[cols])
    return _pair_mlp(diff, params["mlp"], params["m_w"], params["m_b"],
                     params["m2_w"], params["m2_b"])


# speedup vs baseline: 35.9028x; 1.3159x over previous
import functools

import jax
import jax.numpy as jnp
from jax import lax
from jax.experimental import pallas as pl
from jax.experimental.pallas import tpu as pltpu
from jax.experimental.pallas import tpu_sc as plsc

N1 = 10000
N2 = 10000
E = 160000
K_NEAR = 10

_NP = 10048
_EPG = 172032
_CH = _EPG // 32
_NB = 4
_NQ = _CH // (16 * _NB)
_RPT = _NP // 16
_B = 1256


def _make_edge_kernel(F, FA):
    mesh = plsc.VectorSubcoreMesh(core_axis_name="c", subcore_axis_name="s")

    @functools.partial(
        pl.kernel,
        out_type=jax.ShapeDtypeStruct((2, _NP, FA), jnp.float32),
        mesh=mesh,
        scratch_types=[
            pltpu.VMEM((_CH,), jnp.int32),
            pltpu.VMEM((_CH,), jnp.int32),
            pltpu.VMEM((_NP,), jnp.float32),
            pltpu.VMEM((_NP,), jnp.float32),
        ] + [pltpu.VMEM((16, F), jnp.float32) for _ in range(_NB)]
          + [pltpu.VMEM((16, FA), jnp.float32) for _ in range(_NB)]
          + [pltpu.VMEM_SHARED((_NP, FA), jnp.float32)]
          + [pltpu.SemaphoreType.DMA for _ in range(_NB)],
        compiler_params=pltpu.CompilerParams(needs_layout_passes=False,
                                             use_tc_tiling_on_sc=False),
    )
    def k(h_hbm, asrc_hbm, adst_hbm, src_hbm, dst_hbm, zeros_hbm, acc_hbm,
          src_v, dst_v, asrc_v, adst_v, *bufs_flat):
        rows = bufs_flat[:_NB]
        sbuf = bufs_flat[_NB:2 * _NB]
        acc_sh = bufs_flat[2 * _NB]
        gsem = bufs_flat[2 * _NB + 1:]
        cid = lax.axis_index("c")
        sid = lax.axis_index("s")
        base = (cid * 16 + sid) * _CH
        pltpu.sync_copy(src_hbm.at[pl.ds(base, _CH)], src_v)
        pltpu.sync_copy(dst_hbm.at[pl.ds(base, _CH)], dst_v)
        pltpu.sync_copy(asrc_hbm, asrc_v)
        pltpu.sync_copy(adst_hbm, adst_v)
        ro = sid * _RPT
        pltpu.sync_copy(zeros_hbm.at[pl.ds(ro, _RPT)], acc_sh.at[pl.ds(ro, _RPT)])
        plsc.subcore_barrier()

        lane0 = lax.iota(jnp.int32, 16) == 0
        ngr = _CH // 16

        for b in range(_NB):
            i_s = src_v[pl.ds(b * 16, 16)]
            pltpu.async_copy(h_hbm.at[i_s], rows[b], gsem[b])

        def quad_body(qi, carry):
            g0 = qi * _NB
            for b in range(_NB):
                g = g0 + b
                i_s = src_v[pl.ds(g * 16, 16)]
                i_d = dst_v[pl.ds(g * 16, 16)]
                a = plsc.load_gather(asrc_v, [i_s]) + plsc.load_gather(adst_v, [i_d])
                alpha = jnp.where(a >= 0.0, a, 0.2 * a)
                w = jnp.exp(alpha)
                pltpu.make_async_copy(h_hbm.at[i_s], rows[b], gsem[b]).wait()
                for e in range(16):
                    we = w[e]
                    for q in range(F // 16):
                        sbuf[b][e, pl.ds(q * 16, 16)] = rows[b][e, pl.ds(q * 16, 16)] * we
                    sbuf[b][e, pl.ds(F, 16)] = jnp.where(lane0, we, 0.0)
                pltpu.sync_copy(sbuf[b], acc_sh.at[i_d], add=True)
                gp = lax.rem(g + _NB, ngr)
                i_sp = src_v[pl.ds(gp * 16, 16)]
                pltpu.async_copy(h_hbm.at[i_sp], rows[b], gsem[b])
            return carry

        lax.fori_loop(0, _NQ, quad_body, 0)
        for b in range(_NB):
            i_s = src_v[pl.ds(b * 16, 16)]
            pltpu.make_async_copy(h_hbm.at[i_s], rows[b], gsem[b]).wait()
        plsc.subcore_barrier()
        pltpu.sync_copy(acc_sh.at[pl.ds(ro, _RPT)], acc_hbm.at[cid, pl.ds(ro, _RPT)])

    return k


_edge64 = _make_edge_kernel(64, 80)
_edge16 = _make_edge_kernel(16, 32)

_PP = 100352
_CHP = _PP // 32
_NQP = _CHP // (16 * _NB)


def _make_pair_diff():
    mesh = plsc.VectorSubcoreMesh(core_axis_name="c", subcore_axis_name="s")

    @functools.partial(
        pl.kernel,
        out_type=jax.ShapeDtypeStruct((_PP, 32), jnp.float32),
        mesh=mesh,
        scratch_types=[
            pltpu.VMEM((_CHP,), jnp.int32),
            pltpu.VMEM((_CHP,), jnp.int32),
        ] + [pltpu.VMEM((16, 32), jnp.float32) for _ in range(3 * _NB)]
          + [pltpu.SemaphoreType.DMA for _ in range(3 * _NB)],
        compiler_params=pltpu.CompilerParams(needs_layout_passes=False,
                                             use_tc_tiling_on_sc=False),
    )
    def k(pred1_hbm, pred2_hbm, rows_hbm, cols_hbm, diff_hbm, ri_v, ci_v,
          *bufs_flat):
        p1 = bufs_flat[:_NB]
        p2 = bufs_flat[_NB:2 * _NB]
        db = bufs_flat[2 * _NB:3 * _NB]
        g1s = bufs_flat[3 * _NB:4 * _NB]
        g2s = bufs_flat[4 * _NB:5 * _NB]
        oss = bufs_flat[5 * _NB:6 * _NB]
        cid = lax.axis_index("c")
        sid = lax.axis_index("s")
        base = (cid * 16 + sid) * _CHP
        pltpu.sync_copy(rows_hbm.at[pl.ds(base, _CHP)], ri_v)
        pltpu.sync_copy(cols_hbm.at[pl.ds(base, _CHP)], ci_v)
        ngr = _CHP // 16

        for b in range(_NB):
            pltpu.async_copy(pred1_hbm.at[ri_v[pl.ds(b * 16, 16)]], p1[b], g1s[b])
            pltpu.async_copy(pred2_hbm.at[ci_v[pl.ds(b * 16, 16)]], p2[b], g2s[b])

        def quad_body(qi, carry):
            g0 = qi * _NB
            for b in range(_NB):
                g = g0 + b
                i_r = ri_v[pl.ds(g * 16, 16)]
                i_c = ci_v[pl.ds(g * 16, 16)]
                pltpu.make_async_copy(pred1_hbm.at[i_r], p1[b], g1s[b]).wait()
                pltpu.make_async_copy(pred2_hbm.at[i_c], p2[b], g2s[b]).wait()

                @pl.when(qi > 0)
                def _():
                    pltpu.make_async_copy(
                        db[b], diff_hbm.at[pl.ds(base, 16)], oss[b]).wait()

                for e in range(16):
                    for q in range(2):
                        db[b][e, pl.ds(q * 16, 16)] = jnp.abs(
                            p1[b][e, pl.ds(q * 16, 16)] - p2[b][e, pl.ds(q * 16, 16)])
                pltpu.async_copy(db[b], diff_hbm.at[pl.ds(base + g * 16, 16)], oss[b])
                gp = lax.rem(g + _NB, ngr)
                pltpu.async_copy(pred1_hbm.at[ri_v[pl.ds(gp * 16, 16)]], p1[b], g1s[b])
                pltpu.async_copy(pred2_hbm.at[ci_v[pl.ds(gp * 16, 16)]], p2[b], g2s[b])
            return carry

        lax.fori_loop(0, _NQP, quad_body, 0)
        for b in range(_NB):
            pltpu.make_async_copy(pred1_hbm.at[ri_v[pl.ds(b * 16, 16)]], p1[b], g1s[b]).wait()
            pltpu.make_async_copy(pred2_hbm.at[ci_v[pl.ds(b * 16, 16)]], p2[b], g2s[b]).wait()
            pltpu.make_async_copy(db[b], diff_hbm.at[pl.ds(base, 16)], oss[b]).wait()

    return k


_pair_diff = _make_pair_diff()


def _dot(a, b):
    return lax.dot_general(a, b, (((1,), (0,)), ((), ())),
                           preferred_element_type=jnp.float32)


def _mm_att_body(x_ref, w_ref, as_ref, ad_ref, h_ref, s_ref, d_ref):
    h = _dot(x_ref[...], w_ref[...])
    h_ref[...] = h
    s_ref[...] = _dot(h, as_ref[...])
    d_ref[...] = _dot(h, ad_ref[...])


def _mm_att(x, W, att_s, att_d):
    fin, F = W.shape
    h, a_s, a_d = pl.pallas_call(
        _mm_att_body,
        grid=(_NP // _B,),
        in_specs=[
            pl.BlockSpec((_B, fin), lambda i: (i, 0)),
            pl.BlockSpec((fin, F), lambda i: (0, 0)),
            pl.BlockSpec((F, 1), lambda i: (0, 0)),
            pl.BlockSpec((F, 1), lambda i: (0, 0)),
        ],
        out_specs=[
            pl.BlockSpec((_B, F), lambda i: (i, 0)),
            pl.BlockSpec((_B, 1), lambda i: (i, 0)),
            pl.BlockSpec((_B, 1), lambda i: (i, 0)),
        ],
        out_shape=[
            jax.ShapeDtypeStruct((_NP, F), jnp.float32),
            jax.ShapeDtypeStruct((_NP, 1), jnp.float32),
            jax.ShapeDtypeStruct((_NP, 1), jnp.float32),
        ],
    )(x, W, att_s, att_d)
    return h, a_s.reshape(_NP), a_d.reshape(_NP)


def _combine(acc, bias_p, F_tbl):
    FA = acc.shape[2]

    def body(acc_ref, b_ref, out_ref):
        a = acc_ref[0] + acc_ref[1]
        out_ref[...] = a[:, :F_tbl] / a[:, F_tbl:F_tbl + 1] + b_ref[...]

    return pl.pallas_call(
        body,
        grid=(_NP // _B,),
        in_specs=[
            pl.BlockSpec((2, _B, FA), lambda i: (0, i, 0)),
            pl.BlockSpec((1, F_tbl), lambda i: (0, 0)),
        ],
        out_specs=pl.BlockSpec((_B, F_tbl), lambda i: (i, 0)),
        out_shape=jax.ShapeDtypeStruct((_NP, F_tbl), jnp.float32),
    )(acc, bias_p)


def _final_body(x1_ref, w1_ref, b1_ref, x2_ref, w2_ref, b2_ref, out_ref):
    emb = _dot(x1_ref[...], w1_ref[...]) + b1_ref[...]
    emb2 = _dot(x2_ref[...], w2_ref[...]) + b2_ref[...]
    out_ref[...] = jnp.concatenate([emb2, emb], axis=1)


def _final(xc1, w1p, b1, xc2, w2p, b2):
    return pl.pallas_call(
        _final_body,
        grid=(_NP // _B,),
        in_specs=[
            pl.BlockSpec((_B, 384), lambda i: (i, 0)),
            pl.BlockSpec((384, 16), lambda i: (0, 0)),
            pl.BlockSpec((1, 16), lambda i: (0, 0)),
            pl.BlockSpec((_B, 32), lambda i: (i, 0)),
            pl.BlockSpec((32, 16), lambda i: (0, 0)),
            pl.BlockSpec((1, 16), lambda i: (0, 0)),
        ],
        out_specs=pl.BlockSpec((_B, 32), lambda i: (i, 0)),
        out_shape=jax.ShapeDtypeStruct((_NP, 32), jnp.float32),
    )(xc1, w1p, b1, xc2, w2p, b2)


_ROWS = 256
_COLS_PAD = 10112
_N1_PAD = 10240


def _knn_body(x1_ref, x2t_ref, out_ref, d_ref):
    a = x1_ref[...]
    b = x2t_ref[...]
    ab = _dot(a, b)
    aa = jnp.sum(a * a, axis=1, keepdims=True)
    bb = jnp.sum(b * b, axis=0, keepdims=True)
    d = jnp.sqrt(jnp.maximum(aa + bb - 2.0 * ab, 0.0))
    cols = lax.broadcasted_iota(jnp.int32, (_ROWS, _COLS_PAD), 1)
    d_ref[...] = jnp.where(cols >= N2, jnp.inf, d)
    for k in range(K_NEAR):
        dcur = d_ref[...]
        m = jnp.min(dcur, axis=1, keepdims=True)
        idx = jnp.min(jnp.where(dcur <= m, cols, jnp.int32(2 ** 30)),
                      axis=1, keepdims=True)
        out_ref[:, k:k + 1] = idx
        d_ref[...] = jnp.where(cols == idx, jnp.inf, dcur)


def _knn(x1, x2):
    a = jnp.zeros((_N1_PAD, 8), jnp.float32).at[:N1, :3].set(x1[:, :3])
    bt = jnp.zeros((8, _COLS_PAD), jnp.float32).at[:3, :N2].set(x2[:, :3].T)
    out = pl.pallas_call(
        _knn_body,
        grid=(_N1_PAD // _ROWS,),
        in_specs=[
            pl.BlockSpec((_ROWS, 8), lambda i: (i, 0)),
            pl.BlockSpec((8, _COLS_PAD), lambda i: (0, 0)),
        ],
        out_specs=pl.BlockSpec((_ROWS, 16), lambda i: (i, 0)),
        out_shape=jax.ShapeDtypeStruct((_N1_PAD, 16), jnp.int32),
        scratch_shapes=[pltpu.VMEM((_ROWS, _COLS_PAD), jnp.float32)],
    )(a, bt)
    return out[:N1, :K_NEAR]


_PBLK = 2048


def _pair_body(d_ref, w1_ref, b1_ref, w2_ref, b2_ref, mw_ref, mb_ref,
               m2w_ref, m2b_ref, out_ref):
    d = d_ref[...]
    acc = jnp.zeros_like(d)
    for j in range(32):
        acc = acc + jnp.maximum(d * w1_ref[0, j] + b1_ref[0, j], 0.0) * w2_ref[0, j]
    t = acc + b2_ref[0, 0]
    t1 = lax.dot_general(t, mw_ref[...], (((1,), (1,)), ((), ())),
                         preferred_element_type=jnp.float32) + mb_ref[...]
    t2 = lax.dot_general(t1, m2w_ref[...], (((1,), (1,)), ((), ())),
                         preferred_element_type=jnp.float32) + m2b_ref[...]
    out_ref[...] = t2


def _pair_mlp(diff, mlp, m_w, m_b, m2_w, m2_b):
    P = diff.shape[0]
    small = lambda r, c: pl.BlockSpec((r, c), lambda i: (0, 0))
    return pl.pallas_call(
        _pair_body,
        grid=(P // _PBLK,),
        in_specs=[
            pl.BlockSpec((_PBLK, 32), lambda i: (i, 0)),
            small(1, 32), small(1, 32), small(1, 32), small(1, 1),
            small(10, 32), small(1, 10), small(2, 10), small(1, 2),
        ],
        out_specs=pl.BlockSpec((_PBLK, 2), lambda i: (i, 0)),
        out_shape=jax.ShapeDtypeStruct((P, 2), jnp.float32),
    )(diff, mlp["W1"].reshape(1, 32), mlp["b1"].reshape(1, 32),
      mlp["W2"].reshape(1, 32), mlp["b2"].reshape(1, 1),
      m_w, m_b.reshape(1, 10), m2_w, m2_b.reshape(1, 2))


def _padw(W, r, c):
    return jnp.zeros((r, c), jnp.float32).at[:W.shape[0], :W.shape[1]].set(W)


def _pada(v, n):
    return jnp.zeros((n, 1), jnp.float32).at[:v.shape[0], 0].set(v)


def _padb(v, n):
    return jnp.zeros((1, n), jnp.float32).at[0, :v.shape[0]].set(v)


def _encode(x, edge_index, params, zeros80, zeros32):
    n = x.shape[0]
    xs = jnp.zeros((_NP, 16), jnp.float32).at[:n, :10].set(x)
    loop = jnp.arange(n, dtype=jnp.int32)
    pad_e = jnp.full((_EPG - (E + n),), 10000, jnp.int32)
    src = jnp.concatenate([edge_index[0], loop, pad_e])
    dst = jnp.concatenate([edge_index[1], loop, pad_e])

    e1 = params["enc1"]["layers"]
    e2 = params["enc2"]["layers"]

    outs1 = []
    h, a_s, a_d = _mm_att(xs, _padw(e1[0]["W"], 16, 64),
                          e1[0]["att_src"].reshape(64, 1),
                          e1[0]["att_dst"].reshape(64, 1))
    for i in range(5):
        acc = _edge64(h, a_s, a_d, src, dst, zeros80)
        out = _combine(acc, e1[i]["bias"].reshape(1, 64), 64)
        outs1.append(out)
        if i < 4:
            h, a_s, a_d = _mm_att(out, e1[i + 1]["W"],
                                  e1[i + 1]["att_src"].reshape(64, 1),
                                  e1[i + 1]["att_dst"].reshape(64, 1))

    outs2 = []
    h, a_s, a_d = _mm_att(xs, _padw(e2[0]["W"], 16, 16),
                          _pada(e2[0]["att_src"], 16), _pada(e2[0]["att_dst"], 16))
    for i in range(2):
        acc = _edge16(h, a_s, a_d, src, dst, zeros32)
        out = _combine(acc, _padb(e2[i]["bias"], 16), 16)
        outs2.append(out)
        if i < 1:
            h, a_s, a_d = _mm_att(out, _padw(e2[1]["W"], 16, 16),
                                  _pada(e2[1]["att_src"], 16),
                                  _pada(e2[1]["att_dst"], 16))

    xc1 = jnp.concatenate([xs[:, :10]] + outs1
                          + [jnp.zeros((_NP, 54), jnp.float32)], axis=1)
    w1p = jnp.zeros((384, 16), jnp.float32).at[:321].set(
        params["enc1"]["conv_w"][:, 0, :].T)
    xc2 = jnp.concatenate([xs[:, :10], outs2[0][:, :8], outs2[1][:, :8],
                           jnp.zeros((_NP, 6), jnp.float32)], axis=1)
    w2p = jnp.zeros((32, 16), jnp.float32).at[:25].set(
        params["enc2"]["conv_w"][:, 0, :].T)
    return _final(xc1, w1p, params["enc1"]["conv_b"].reshape(1, 16),
                  xc2, w2p, params["enc2"]["conv_b"].reshape(1, 16))


def kernel(x1, edge_attr1, x2, edge_attr2, params, edge_index1, edge_index2):
    zeros80 = jnp.zeros((_NP, 80), jnp.float32)
    zeros32 = jnp.zeros((_NP, 32), jnp.float32)
    pred1 = _encode(x1, edge_index1, params, zeros80, zeros32)
    pred2 = _encode(x2, edge_index2, params, zeros80, zeros32)

    nearest = _knn(x1, x2)
    pad_p = jnp.full((_PP - N1 * K_NEAR,), 10000, jnp.int32)
    cols = jnp.concatenate([jnp.sort(nearest, axis=1).reshape(-1), pad_p])
    rows = jnp.concatenate(
        [jnp.repeat(jnp.arange(N1, dtype=jnp.int32), K_NEAR), pad_p])
    diff = _pair_diff(pred1, pred2, rows, cols)
    out = _pair_mlp(diff, params["mlp"], params["m_w"], params["m_b"],
                    params["m2_w"], params["m2_b"])
    return out[:N1 * K_NEAR]
